# baseline (speedup 1.0000x reference)
import functools

import jax
import jax.numpy as jnp
from jax.experimental import pallas as pl
from jax.experimental.pallas import tpu as pltpu

_STRIDE = 8.0
_IMG_W = 512.0
_IMG_H = 512.0
_NCLS = 80
_P = 1024


def _fcos_kernel(bbox_ref, conf_ref, cls_ref, obb_ref, oidx_ref, osc_ref,
                 *, ncols):
    j = pl.program_id(1)
    b = bbox_ref[0]
    cf = conf_ref[0]
    c = cls_ref[0]

    pix = jax.lax.broadcasted_iota(jnp.int32, (_P, 1), 0) + j * _P
    shift = ncols.bit_length() - 1
    xc = (pix & (ncols - 1)).astype(jnp.float32) * _STRIDE + _STRIDE / 2.0
    yc = (pix >> shift).astype(jnp.float32) * _STRIDE + _STRIDE / 2.0

    p = jnp.exp(b) * _STRIDE
    x1 = jnp.clip(xc - p[:, 0:1], 0.0, _IMG_W)
    y1 = jnp.clip(yc - p[:, 1:2], 0.0, _IMG_H)
    x2 = jnp.clip(xc + p[:, 2:3], 0.0, _IMG_W)
    y2 = jnp.clip(yc + p[:, 3:4], 0.0, _IMG_H)
    obb_ref[0] = jnp.concatenate(
        [(x1 + x2) * 0.5, (y1 + y2) * 0.5, x2 - x1, y2 - y1], axis=1)

    m = jnp.max(c, axis=1, keepdims=True)
    lane = jax.lax.broadcasted_iota(jnp.int32, (_P, _NCLS), 1)
    idx = jnp.min(jnp.where(c == m, lane, _NCLS), axis=1, keepdims=True)
    oidx_ref[0] = idx
    osc_ref[0] = jnp.sqrt(jax.nn.sigmoid(cf) * jax.nn.sigmoid(m))


def kernel(bbox, conf, cls):
    nB, nH, nW, _ = bbox.shape
    npix = nH * nW
    bbox_r = bbox.reshape(nB, npix, 4)
    conf_r = conf.reshape(nB, npix, 1)
    cls_r = cls.reshape(nB, npix, _NCLS)

    grid = (nB, npix // _P)
    out_shapes = (
        jax.ShapeDtypeStruct((nB, npix, 4), jnp.float32),
        jax.ShapeDtypeStruct((nB, npix, 1), jnp.int32),
        jax.ShapeDtypeStruct((nB, npix, 1), jnp.float32),
    )
    obb, oidx, osc = pl.pallas_call(
        functools.partial(_fcos_kernel, ncols=nW),
        grid=grid,
        in_specs=[
            pl.BlockSpec((1, _P, 4), lambda i, j: (i, j, 0)),
            pl.BlockSpec((1, _P, 1), lambda i, j: (i, j, 0)),
            pl.BlockSpec((1, _P, _NCLS), lambda i, j: (i, j, 0)),
        ],
        out_specs=(
            pl.BlockSpec((1, _P, 4), lambda i, j: (i, j, 0)),
            pl.BlockSpec((1, _P, 1), lambda i, j: (i, j, 0)),
            pl.BlockSpec((1, _P, 1), lambda i, j: (i, j, 0)),
        ),
        out_shape=out_shapes,
        compiler_params=pltpu.CompilerParams(
            dimension_semantics=("parallel", "parallel")),
    )(bbox_r, conf_r, cls_r)
    return (obb, oidx.reshape(nB, npix), osc.reshape(nB, npix))


# speedup vs baseline: 3.0139x; 3.0139x over previous
import jax
import jax.numpy as jnp
from jax.experimental import pallas as pl
from jax.experimental.pallas import tpu as pltpu

_STRIDE = 8.0
_IMG_W = 512.0
_IMG_H = 512.0
_NCLS = 80


def _fcos_kernel(bbox_ref, conf_ref, cls_ref, obb_ref, oidx_ref, osc_ref):
    b = bbox_ref[0]
    fr = jax.lax.broadcasted_iota(jnp.int32, (128, 128), 0)
    fl = jax.lax.broadcasted_iota(jnp.int32, (128, 128), 1)
    flat = fr * 128 + fl
    pix = flat >> 2
    chan = flat & 3
    xc = (pix & 63).astype(jnp.float32) * _STRIDE + _STRIDE / 2.0
    yc = ((pix >> 6) & 63).astype(jnp.float32) * _STRIDE + _STRIDE / 2.0
    ctr = jnp.where((chan & 1) == 0, xc, yc)
    sgn = jnp.where(chan < 2, -1.0, 1.0)
    e = jnp.clip(ctr + sgn * (jnp.exp(b) * _STRIDE), 0.0, _IMG_W)
    el = pltpu.roll(e, 126, 1)
    er = pltpu.roll(e, 2, 1)
    obb_ref[0] = jnp.where(chan < 2, (e + el) * 0.5, e - er)

    c = cls_ref[0]
    m = jnp.max(c, axis=1, keepdims=True)
    lane = jax.lax.broadcasted_iota(jnp.int32, (4096, _NCLS), 1)
    idx = jnp.min(jnp.where(c == m, lane, _NCLS), axis=1, keepdims=True)
    m2 = m.reshape(32, 128)
    oidx_ref[0] = idx.reshape(32, 128)
    osc_ref[0] = jnp.sqrt(jax.nn.sigmoid(conf_ref[0]) * jax.nn.sigmoid(m2))


def kernel(bbox, conf, cls):
    nB, nH, nW, _ = bbox.shape
    npix = nH * nW
    bbox_r = bbox.reshape(nB, 128, 128)
    conf_r = conf.reshape(nB, 32, 128)
    cls_r = cls.reshape(nB, npix, _NCLS)

    out_shapes = (
        jax.ShapeDtypeStruct((nB, 128, 128), jnp.float32),
        jax.ShapeDtypeStruct((nB, 32, 128), jnp.int32),
        jax.ShapeDtypeStruct((nB, 32, 128), jnp.float32),
    )
    obb, oidx, osc = pl.pallas_call(
        _fcos_kernel,
        grid=(nB,),
        in_specs=[
            pl.BlockSpec((1, 128, 128), lambda i: (i, 0, 0)),
            pl.BlockSpec((1, 32, 128), lambda i: (i, 0, 0)),
            pl.BlockSpec((1, npix, _NCLS), lambda i: (i, 0, 0)),
        ],
        out_specs=(
            pl.BlockSpec((1, 128, 128), lambda i: (i, 0, 0)),
            pl.BlockSpec((1, 32, 128), lambda i: (i, 0, 0)),
            pl.BlockSpec((1, 32, 128), lambda i: (i, 0, 0)),
        ),
        out_shape=out_shapes,
        compiler_params=pltpu.CompilerParams(
            dimension_semantics=("parallel",)),
    )(bbox_r, conf_r, cls_r)
    return (obb.reshape(nB, npix, 4), oidx.reshape(nB, npix),
            osc.reshape(nB, npix))
